# baseline (speedup 1.0000x reference)
import functools

import jax
import jax.numpy as jnp
from jax import lax
from jax.experimental import pallas as pl
from jax.experimental.pallas import tpu as pltpu
from jax.experimental.pallas import tpu_sc as plsc

E = 8
TOPK = 2
H = 2048
FF = 1408
T = 2048
TS = T * TOPK
BM = 64
NB = TS // BM + E
NBT = NB * BM
NBPAD = 80
BT = 256
MROW = 32
MCOL = 128

NTILES = 32
TOK_W = T // NTILES
CH_S = 16
CH_T = 8


@functools.cache
def _sc_mesh():
    return plsc.VectorSubcoreMesh(core_axis_name="c", subcore_axis_name="s",
                                  num_cores=2, num_subcores=16)


def _router_body(x_ref, gw_ref, idx_ref, wrep_ref):
    x = x_ref[...]
    logits = lax.dot_general(x, gw_ref[...], (((1,), (1,)), ((), ())),
                             preferred_element_type=jnp.float32)
    m = jnp.max(logits, axis=1, keepdims=True)
    ex = jnp.exp(logits - m)
    rw = ex / jnp.sum(ex, axis=1, keepdims=True)
    iota = lax.broadcasted_iota(jnp.int32, rw.shape, 1)
    a1 = jnp.argmax(rw, axis=1, keepdims=True).astype(jnp.int32)
    s1 = jnp.max(rw, axis=1, keepdims=True)
    rw2 = jnp.where(iota == a1, -1.0, rw)
    a2 = jnp.argmax(rw2, axis=1, keepdims=True).astype(jnp.int32)
    s2 = jnp.max(rw2, axis=1, keepdims=True)
    denom = s1 + s2 + 1e-20
    idx_ref[...] = jnp.concatenate([a1, a2], axis=1)
    wrep_ref[...] = jnp.concatenate(
        [jnp.broadcast_to(s1 / denom, (s1.shape[0], 16)),
         jnp.broadcast_to(s2 / denom, (s2.shape[0], 16))], axis=1)


def _router(x, gate_weight):
    return pl.pallas_call(
        _router_body,
        grid=(T // BT,),
        in_specs=[
            pl.BlockSpec((BT, H), lambda t: (t, 0)),
            pl.BlockSpec((E, H), lambda t: (0, 0)),
        ],
        out_specs=[
            pl.BlockSpec((BT, TOPK), lambda t: (t, 0)),
            pl.BlockSpec((BT, 32), lambda t: (t, 0)),
        ],
        out_shape=[
            jax.ShapeDtypeStruct((T, TOPK), jnp.int32),
            jax.ShapeDtypeStruct((T, 32), jnp.float32),
        ],
    )(x, gate_weight)


def _tc_meta_body(ids_ref, dest_ref, be_ref):
    ids = ids_ref[...]
    li = lax.broadcasted_iota(jnp.int32, (MCOL, MCOL), 0)
    lj = lax.broadcasted_iota(jnp.int32, (MCOL, MCOL), 1)
    lt_incl = (li <= lj).astype(jnp.float32)
    ri = lax.broadcasted_iota(jnp.int32, (MROW, MROW), 0)
    rj = lax.broadcasted_iota(jnp.int32, (MROW, MROW), 1)
    tri_strict = (rj < ri).astype(jnp.float32)
    ones_col = jnp.ones((MCOL, MCOL), jnp.float32)

    ranks = []
    counts = []
    for e in range(E):
        oh = (ids == e).astype(jnp.float32)
        pre = lax.dot_general(oh, lt_incl, (((1,), (0,)), ((), ())),
                              preferred_element_type=jnp.float32)
        rowtot = lax.dot_general(oh, ones_col, (((1,), (0,)), ((), ())),
                                 preferred_element_type=jnp.float32)
        crosspre = lax.dot_general(tri_strict, rowtot,
                                   (((1,), (0,)), ((), ())),
                                   preferred_element_type=jnp.float32)
        ranks.append(pre - oh + crosspre)
        counts.append(jnp.sum(oh))

    bs_excl = []
    acc = 0.0
    for e in range(E):
        bs_excl.append(acc)
        acc = acc + jnp.ceil(counts[e] * (1.0 / BM))
    dest = jnp.zeros((MROW, MCOL), jnp.float32)
    for e in range(E):
        oh = (ids == e).astype(jnp.float32)
        dest = dest + oh * (ranks[e] + bs_excl[e] * BM)
    dest_ref[...] = dest.astype(jnp.int32)

    j = lax.broadcasted_iota(jnp.int32, (1, NBPAD), 1)
    be = jnp.zeros((1, NBPAD), jnp.int32)
    for e in range(1, E):
        be = be + (j >= bs_excl[e].astype(jnp.int32)).astype(jnp.int32)
    be_ref[...] = be


def _tc_meta(ids2d):
    return pl.pallas_call(
        _tc_meta_body,
        grid=(1,),
        in_specs=[pl.BlockSpec((MROW, MCOL), lambda i: (0, 0))],
        out_specs=[
            pl.BlockSpec((MROW, MCOL), lambda i: (0, 0)),
            pl.BlockSpec((1, NBPAD), lambda i: (0, 0)),
        ],
        out_shape=[
            jax.ShapeDtypeStruct((MROW, MCOL), jnp.int32),
            jax.ShapeDtypeStruct((1, NBPAD), jnp.int32),
        ],
    )(ids2d)


NCH_S = TOK_W // CH_S


def _dispatch_body(x_hbm, d0_hbm, d1_hbm, xs_hbm,
                   buf0, buf1, i00, i01, i10, i11,
                   s00, s01, s10, s11):
    wid = lax.axis_index("s") * 2 + lax.axis_index("c")
    tbase = wid * TOK_W
    bufs = (buf0, buf1)
    idx0 = (i00, i01)
    idx1 = (i10, i11)
    sem0 = (s00, s01)
    sem1 = (s10, s11)

    sd = [None] * NCH_S
    for c in range(NCH_S):
        b = c % 2
        if c >= 2:
            for d in sd[c - 2]:
                d.wait()
        tok = tbase + c * CH_S
        pltpu.sync_copy(d0_hbm.at[pl.ds(tok, CH_S)], idx0[b])
        pltpu.sync_copy(d1_hbm.at[pl.ds(tok, CH_S)], idx1[b])
        pltpu.sync_copy(x_hbm.at[pl.ds(tok, CH_S)], bufs[b])
        sd[c] = (
            pltpu.async_copy(bufs[b], xs_hbm.at[idx0[b]], sem0[b]),
            pltpu.async_copy(bufs[b], xs_hbm.at[idx1[b]], sem1[b]),
        )
    for c in range(max(0, NCH_S - 2), NCH_S):
        for d in sd[c]:
            d.wait()


def _sc_dispatch(x, d0, d1):
    return pl.kernel(
        _dispatch_body,
        out_type=jax.ShapeDtypeStruct((NBT, H), jnp.float32),
        mesh=_sc_mesh(),
        scratch_types=(
            pltpu.VMEM((CH_S, H), jnp.float32),
            pltpu.VMEM((CH_S, H), jnp.float32),
            pltpu.VMEM((CH_S,), jnp.int32),
            pltpu.VMEM((CH_S,), jnp.int32),
            pltpu.VMEM((CH_S,), jnp.int32),
            pltpu.VMEM((CH_S,), jnp.int32),
            pltpu.SemaphoreType.DMA,
            pltpu.SemaphoreType.DMA,
            pltpu.SemaphoreType.DMA,
            pltpu.SemaphoreType.DMA,
        ),
    )(x, d0, d1)


NCH_T = TOK_W // CH_T


def _combine_body(ys_hbm, dest_hbm, wrep_hbm, out_hbm, didx_v, wv,
                  rbuf0, rbuf1, obuf0, obuf1, gsem0, gsem1, ssem0, ssem1):
    wid = lax.axis_index("s") * 2 + lax.axis_index("c")
    base = wid * TOK_W
    pltpu.sync_copy(dest_hbm.at[pl.ds(base * 2, TOK_W * 2)], didx_v)
    pltpu.sync_copy(wrep_hbm.at[pl.ds(base, TOK_W)], wv)
    rbufs = (rbuf0, rbuf1)
    obufs = (obuf0, obuf1)
    gsems = (gsem0, gsem1)
    ssems = (ssem0, ssem1)

    gd = [None] * NCH_T
    sd = [None] * NCH_T
    gd[0] = pltpu.async_copy(ys_hbm.at[didx_v.at[pl.ds(0, 2 * CH_T)]],
                             rbufs[0], gsems[0])
    for c in range(NCH_T):
        b = c % 2
        gd[c].wait()
        if c + 1 < NCH_T:
            b2 = (c + 1) % 2
            gd[c + 1] = pltpu.async_copy(
                ys_hbm.at[didx_v.at[pl.ds((c + 1) * 2 * CH_T, 2 * CH_T)]],
                rbufs[b2], gsems[b2])
        if c >= 2:
            sd[c - 2].wait()
        rbuf = rbufs[b]
        obuf = obufs[b]

        def col(cc, inner):
            sl = pl.ds(cc * 16, 16)
            for j in range(CH_T):
                w0 = wv[c * CH_T + j, pl.ds(0, 16)]
                w1 = wv[c * CH_T + j, pl.ds(16, 16)]
                obuf[j, sl] = (w0 * rbuf[2 * j, sl]
                               + w1 * rbuf[2 * j + 1, sl])
            return inner

        lax.fori_loop(0, H // 16, col, 0)
        sd[c] = pltpu.async_copy(obuf,
                                 out_hbm.at[pl.ds(base + c * CH_T, CH_T)],
                                 ssems[b])
    sd[NCH_T - 1].wait()
    sd[NCH_T - 2].wait()


def _sc_combine(ys, dest, wrep):
    return pl.kernel(
        _combine_body,
        out_type=jax.ShapeDtypeStruct((T, H), jnp.float32),
        mesh=_sc_mesh(),
        scratch_types=(
            pltpu.VMEM((2 * TOK_W,), jnp.int32),
            pltpu.VMEM((TOK_W, 32), jnp.float32),
            pltpu.VMEM((2 * CH_T, H), jnp.float32),
            pltpu.VMEM((2 * CH_T, H), jnp.float32),
            pltpu.VMEM((CH_T, H), jnp.float32),
            pltpu.VMEM((CH_T, H), jnp.float32),
            pltpu.SemaphoreType.DMA,
            pltpu.SemaphoreType.DMA,
            pltpu.SemaphoreType.DMA,
            pltpu.SemaphoreType.DMA,
        ),
    )(ys, dest, wrep)


def _k1_body(be_ref, xs_ref, gp_ref, up_ref, h_ref):
    x = xs_ref[...]
    g = lax.dot_general(x, gp_ref[0], (((1,), (1,)), ((), ())),
                        preferred_element_type=jnp.float32)
    u = lax.dot_general(x, up_ref[0], (((1,), (1,)), ((), ())),
                        preferred_element_type=jnp.float32)
    h_ref[...] = ((g * jax.nn.sigmoid(g)) * u).astype(jnp.bfloat16)


def _k1(be, xs, gate_proj, up_proj):
    return pl.pallas_call(
        _k1_body,
        grid_spec=pltpu.PrefetchScalarGridSpec(
            num_scalar_prefetch=1,
            grid=(NB,),
            in_specs=[
                pl.BlockSpec((BM, H), lambda b, be: (b, 0)),
                pl.BlockSpec((1, FF, H), lambda b, be: (be[b], 0, 0)),
                pl.BlockSpec((1, FF, H), lambda b, be: (be[b], 0, 0)),
            ],
            out_specs=pl.BlockSpec((BM, FF), lambda b, be: (b, 0)),
        ),
        out_shape=jax.ShapeDtypeStruct((NBT, FF), jnp.bfloat16),
    )(be, xs, gate_proj, up_proj)


def _k2_body(be_ref, h_ref, dp_ref, y_ref):
    y_ref[...] = lax.dot_general(h_ref[...], dp_ref[0],
                                 (((1,), (1,)), ((), ())),
                                 preferred_element_type=jnp.float32)


def _k2(be, hmid, down_proj):
    return pl.pallas_call(
        _k2_body,
        grid_spec=pltpu.PrefetchScalarGridSpec(
            num_scalar_prefetch=1,
            grid=(NB,),
            in_specs=[
                pl.BlockSpec((BM, FF), lambda b, be: (b, 0)),
                pl.BlockSpec((1, H, FF), lambda b, be: (be[b], 0, 0)),
            ],
            out_specs=pl.BlockSpec((BM, H), lambda b, be: (b, 0)),
        ),
        out_shape=jax.ShapeDtypeStruct((NBT, H), jnp.float32),
    )(be, hmid, down_proj)


@jax.jit
def kernel(hidden_states, gate_weight, gate_proj, up_proj, down_proj):
    bsz, seq, hdim = hidden_states.shape
    x = hidden_states.reshape(-1, hdim)

    topk_idx, wrep = _router(x, gate_weight)
    dest2d, be2d = _tc_meta(topk_idx.reshape(MROW, MCOL))
    dest = dest2d.reshape(TS)
    be = be2d.reshape(NBPAD)
    dpair = dest.reshape(T, TOPK)
    xs = _sc_dispatch(x, dpair[:, 0], dpair[:, 1])
    hmid = _k1(be, xs, gate_proj, up_proj)
    ys = _k2(be, hmid, down_proj)
    out = _sc_combine(ys, dest, wrep)
    return out.reshape(bsz, seq, hdim)


# speedup vs baseline: 1.3528x; 1.3528x over previous
import functools

import jax
import jax.numpy as jnp
from jax import lax
from jax.experimental import pallas as pl
from jax.experimental.pallas import tpu as pltpu
from jax.experimental.pallas import tpu_sc as plsc

E = 8
TOPK = 2
H = 2048
FF = 1408
T = 2048
TS = T * TOPK
BM = 128
NB = TS // BM + E
NBT = NB * BM
NBPAD = 48
BT = 256
MROW = 32
MCOL = 128

NTILES = 32
TOK_W = T // NTILES
CH_S = 16
CH_T = 8


@functools.cache
def _sc_mesh():
    return plsc.VectorSubcoreMesh(core_axis_name="c", subcore_axis_name="s",
                                  num_cores=2, num_subcores=16)


def _router_body(x_ref, gw_ref, idx_ref, wrep_ref):
    x = x_ref[...]
    logits = lax.dot_general(x, gw_ref[...], (((1,), (1,)), ((), ())),
                             preferred_element_type=jnp.float32)
    m = jnp.max(logits, axis=1, keepdims=True)
    ex = jnp.exp(logits - m)
    rw = ex / jnp.sum(ex, axis=1, keepdims=True)
    iota = lax.broadcasted_iota(jnp.int32, rw.shape, 1)
    a1 = jnp.argmax(rw, axis=1, keepdims=True).astype(jnp.int32)
    s1 = jnp.max(rw, axis=1, keepdims=True)
    rw2 = jnp.where(iota == a1, -1.0, rw)
    a2 = jnp.argmax(rw2, axis=1, keepdims=True).astype(jnp.int32)
    s2 = jnp.max(rw2, axis=1, keepdims=True)
    denom = s1 + s2 + 1e-20
    idx_ref[...] = jnp.concatenate([a1, a2], axis=1)
    wrep_ref[...] = jnp.concatenate(
        [jnp.broadcast_to(s1 / denom, (s1.shape[0], 16)),
         jnp.broadcast_to(s2 / denom, (s2.shape[0], 16))], axis=1)


def _router(x, gate_weight):
    return pl.pallas_call(
        _router_body,
        grid=(T // BT,),
        in_specs=[
            pl.BlockSpec((BT, H), lambda t: (t, 0)),
            pl.BlockSpec((E, H), lambda t: (0, 0)),
        ],
        out_specs=[
            pl.BlockSpec((BT, TOPK), lambda t: (t, 0)),
            pl.BlockSpec((BT, 32), lambda t: (t, 0)),
        ],
        out_shape=[
            jax.ShapeDtypeStruct((T, TOPK), jnp.int32),
            jax.ShapeDtypeStruct((T, 32), jnp.float32),
        ],
    )(x, gate_weight)


def _tc_meta_body(ids_ref, dest_ref, be_ref):
    ids = ids_ref[...]
    li = lax.broadcasted_iota(jnp.int32, (MCOL, MCOL), 0)
    lj = lax.broadcasted_iota(jnp.int32, (MCOL, MCOL), 1)
    lt_incl = (li <= lj).astype(jnp.float32)
    ri = lax.broadcasted_iota(jnp.int32, (MROW, MROW), 0)
    rj = lax.broadcasted_iota(jnp.int32, (MROW, MROW), 1)
    tri_strict = (rj < ri).astype(jnp.float32)
    ones_col = jnp.ones((MCOL, MCOL), jnp.float32)

    ranks = []
    counts = []
    for e in range(E):
        oh = (ids == e).astype(jnp.float32)
        pre = lax.dot_general(oh, lt_incl, (((1,), (0,)), ((), ())),
                              preferred_element_type=jnp.float32)
        rowtot = lax.dot_general(oh, ones_col, (((1,), (0,)), ((), ())),
                                 preferred_element_type=jnp.float32)
        crosspre = lax.dot_general(tri_strict, rowtot,
                                   (((1,), (0,)), ((), ())),
                                   preferred_element_type=jnp.float32)
        ranks.append(pre - oh + crosspre)
        counts.append(jnp.sum(oh))

    bs_excl = []
    acc = 0.0
    for e in range(E):
        bs_excl.append(acc)
        acc = acc + jnp.ceil(counts[e] * (1.0 / BM))
    dest = jnp.zeros((MROW, MCOL), jnp.float32)
    for e in range(E):
        oh = (ids == e).astype(jnp.float32)
        dest = dest + oh * (ranks[e] + bs_excl[e] * BM)
    dest_ref[...] = dest.astype(jnp.int32)

    j = lax.broadcasted_iota(jnp.int32, (1, NBPAD), 1)
    be = jnp.zeros((1, NBPAD), jnp.int32)
    for e in range(1, E):
        be = be + (j >= bs_excl[e].astype(jnp.int32)).astype(jnp.int32)
    be_ref[...] = be


def _tc_meta(ids2d):
    return pl.pallas_call(
        _tc_meta_body,
        grid=(1,),
        in_specs=[pl.BlockSpec((MROW, MCOL), lambda i: (0, 0))],
        out_specs=[
            pl.BlockSpec((MROW, MCOL), lambda i: (0, 0)),
            pl.BlockSpec((1, NBPAD), lambda i: (0, 0)),
        ],
        out_shape=[
            jax.ShapeDtypeStruct((MROW, MCOL), jnp.int32),
            jax.ShapeDtypeStruct((1, NBPAD), jnp.int32),
        ],
    )(ids2d)


NCH_S = TOK_W // CH_S


def _dispatch_body(x_hbm, d0_hbm, d1_hbm, xs_hbm,
                   buf0, buf1, i00, i01, i10, i11,
                   s00, s01, s10, s11):
    wid = lax.axis_index("s") * 2 + lax.axis_index("c")
    tbase = wid * TOK_W
    bufs = (buf0, buf1)
    idx0 = (i00, i01)
    idx1 = (i10, i11)
    sem0 = (s00, s01)
    sem1 = (s10, s11)

    sd = [None] * NCH_S
    for c in range(NCH_S):
        b = c % 2
        if c >= 2:
            for d in sd[c - 2]:
                d.wait()
        tok = tbase + c * CH_S
        pltpu.sync_copy(d0_hbm.at[pl.ds(tok, CH_S)], idx0[b])
        pltpu.sync_copy(d1_hbm.at[pl.ds(tok, CH_S)], idx1[b])
        pltpu.sync_copy(x_hbm.at[pl.ds(tok, CH_S)], bufs[b])
        sd[c] = (
            pltpu.async_copy(bufs[b], xs_hbm.at[idx0[b]], sem0[b]),
            pltpu.async_copy(bufs[b], xs_hbm.at[idx1[b]], sem1[b]),
        )
    for c in range(max(0, NCH_S - 2), NCH_S):
        for d in sd[c]:
            d.wait()


def _sc_dispatch(x, d0, d1):
    return pl.kernel(
        _dispatch_body,
        out_type=jax.ShapeDtypeStruct((NBT, H), jnp.float32),
        mesh=_sc_mesh(),
        scratch_types=(
            pltpu.VMEM((CH_S, H), jnp.float32),
            pltpu.VMEM((CH_S, H), jnp.float32),
            pltpu.VMEM((CH_S,), jnp.int32),
            pltpu.VMEM((CH_S,), jnp.int32),
            pltpu.VMEM((CH_S,), jnp.int32),
            pltpu.VMEM((CH_S,), jnp.int32),
            pltpu.SemaphoreType.DMA,
            pltpu.SemaphoreType.DMA,
            pltpu.SemaphoreType.DMA,
            pltpu.SemaphoreType.DMA,
        ),
    )(x, d0, d1)


NCH_T = TOK_W // CH_T


def _combine_body(ys_hbm, dest_hbm, wrep_hbm, out_hbm, didx_v, wv,
                  rbuf0, rbuf1, obuf0, obuf1, gsem0, gsem1, ssem0, ssem1):
    wid = lax.axis_index("s") * 2 + lax.axis_index("c")
    base = wid * TOK_W
    pltpu.sync_copy(dest_hbm.at[pl.ds(base * 2, TOK_W * 2)], didx_v)
    pltpu.sync_copy(wrep_hbm.at[pl.ds(base, TOK_W)], wv)
    rbufs = (rbuf0, rbuf1)
    obufs = (obuf0, obuf1)
    gsems = (gsem0, gsem1)
    ssems = (ssem0, ssem1)

    gd = [None] * NCH_T
    sd = [None] * NCH_T
    gd[0] = pltpu.async_copy(ys_hbm.at[didx_v.at[pl.ds(0, 2 * CH_T)]],
                             rbufs[0], gsems[0])
    for c in range(NCH_T):
        b = c % 2
        gd[c].wait()
        if c + 1 < NCH_T:
            b2 = (c + 1) % 2
            gd[c + 1] = pltpu.async_copy(
                ys_hbm.at[didx_v.at[pl.ds((c + 1) * 2 * CH_T, 2 * CH_T)]],
                rbufs[b2], gsems[b2])
        if c >= 2:
            sd[c - 2].wait()
        rbuf = rbufs[b]
        obuf = obufs[b]

        def col(cc, inner):
            sl = pl.ds(cc * 16, 16)
            for j in range(CH_T):
                w0 = wv[c * CH_T + j, pl.ds(0, 16)]
                w1 = wv[c * CH_T + j, pl.ds(16, 16)]
                obuf[j, sl] = (w0 * rbuf[2 * j, sl]
                               + w1 * rbuf[2 * j + 1, sl])
            return inner

        lax.fori_loop(0, H // 16, col, 0)
        sd[c] = pltpu.async_copy(obuf,
                                 out_hbm.at[pl.ds(base + c * CH_T, CH_T)],
                                 ssems[b])
    sd[NCH_T - 1].wait()
    sd[NCH_T - 2].wait()


def _sc_combine(ys, dest, wrep):
    return pl.kernel(
        _combine_body,
        out_type=jax.ShapeDtypeStruct((T, H), jnp.float32),
        mesh=_sc_mesh(),
        scratch_types=(
            pltpu.VMEM((2 * TOK_W,), jnp.int32),
            pltpu.VMEM((TOK_W, 32), jnp.float32),
            pltpu.VMEM((2 * CH_T, H), jnp.float32),
            pltpu.VMEM((2 * CH_T, H), jnp.float32),
            pltpu.VMEM((CH_T, H), jnp.float32),
            pltpu.VMEM((CH_T, H), jnp.float32),
            pltpu.SemaphoreType.DMA,
            pltpu.SemaphoreType.DMA,
            pltpu.SemaphoreType.DMA,
            pltpu.SemaphoreType.DMA,
        ),
    )(ys, dest, wrep)


def _k1_body(be_ref, xs_ref, gp_ref, up_ref, h_ref):
    x = xs_ref[...]
    g = lax.dot_general(x, gp_ref[0], (((1,), (1,)), ((), ())),
                        preferred_element_type=jnp.float32)
    u = lax.dot_general(x, up_ref[0], (((1,), (1,)), ((), ())),
                        preferred_element_type=jnp.float32)
    h_ref[...] = ((g * jax.nn.sigmoid(g)) * u).astype(jnp.bfloat16)


def _k1(be, xs, gate_proj, up_proj):
    return pl.pallas_call(
        _k1_body,
        grid_spec=pltpu.PrefetchScalarGridSpec(
            num_scalar_prefetch=1,
            grid=(NB,),
            in_specs=[
                pl.BlockSpec((BM, H), lambda b, be: (b, 0)),
                pl.BlockSpec((1, FF, H), lambda b, be: (be[b], 0, 0)),
                pl.BlockSpec((1, FF, H), lambda b, be: (be[b], 0, 0)),
            ],
            out_specs=pl.BlockSpec((BM, FF), lambda b, be: (b, 0)),
        ),
        out_shape=jax.ShapeDtypeStruct((NBT, FF), jnp.bfloat16),
    )(be, xs, gate_proj, up_proj)


def _k2_body(be_ref, h_ref, dp_ref, y_ref):
    y_ref[...] = lax.dot_general(h_ref[...], dp_ref[0],
                                 (((1,), (1,)), ((), ())),
                                 preferred_element_type=jnp.float32)


def _k2(be, hmid, down_proj):
    return pl.pallas_call(
        _k2_body,
        grid_spec=pltpu.PrefetchScalarGridSpec(
            num_scalar_prefetch=1,
            grid=(NB,),
            in_specs=[
                pl.BlockSpec((BM, FF), lambda b, be: (b, 0)),
                pl.BlockSpec((1, H, FF), lambda b, be: (be[b], 0, 0)),
            ],
            out_specs=pl.BlockSpec((BM, H), lambda b, be: (b, 0)),
        ),
        out_shape=jax.ShapeDtypeStruct((NBT, H), jnp.float32),
    )(be, hmid, down_proj)


@jax.jit
def kernel(hidden_states, gate_weight, gate_proj, up_proj, down_proj):
    bsz, seq, hdim = hidden_states.shape
    x = hidden_states.reshape(-1, hdim)

    topk_idx, wrep = _router(x, gate_weight)
    dest2d, be2d = _tc_meta(topk_idx.reshape(MROW, MCOL))
    dest = dest2d.reshape(TS)
    be = be2d.reshape(NBPAD)
    dpair = dest.reshape(T, TOPK)
    xs = _sc_dispatch(x, dpair[:, 0], dpair[:, 1])
    hmid = _k1(be, xs, gate_proj, up_proj)
    ys = _k2(be, hmid, down_proj)
    out = _sc_combine(ys, dest, wrep)
    return out.reshape(bsz, seq, hdim)
